# Initial kernel scaffold; baseline (speedup 1.0000x reference)
import jax
import jax.numpy as jnp
from jax.experimental import pallas as pl


def kernel(x, edge_index, W_src, W_dst, att_src, att_dst):
    raise NotImplementedError("write your pallas kernel here")


# speedup vs baseline: 12.3556x; 12.3556x over previous
import functools

import jax
import jax.numpy as jnp
from jax import lax
from jax.experimental import pallas as pl
from jax.experimental.pallas import tpu as pltpu
from jax.experimental.pallas import tpu_sc as plsc

F32 = jnp.float32
NC = 2
NS = 16
CHUNK = 128
W = 80


def _proj_kernel(x_ref, ws_ref, wd_ref, as_exp_ref, ad_exp_ref, xs_ref,
                 al_ref, m_ref, v_scr, m_scr):
    i = pl.program_id(0)

    @pl.when(i == 0)
    def _():
        v_scr[...] = jnp.concatenate(
            [jnp.dot(ws_ref[...], as_exp_ref[...],
                     preferred_element_type=F32),
             jnp.dot(wd_ref[...], ad_exp_ref[...],
                     preferred_element_type=F32)], axis=1)
        m_scr[...] = jnp.full(m_scr.shape, -jnp.inf, F32)

    xb = x_ref[...]
    xs = jnp.dot(xb, ws_ref[...], preferred_element_type=F32)
    xs_ref[...] = xs
    al = jnp.dot(xb, v_scr[...], preferred_element_type=F32)
    al_ref[...] = al
    m_scr[...] = jnp.maximum(m_scr[...], jnp.max(al, axis=0, keepdims=True))
    m_ref[...] = m_scr[...]


def _scale_kernel(xs_ref, al_ref, m_ref, y_ref, ap_ref):
    nch = y_ref.shape[0]
    b = xs_ref.shape[0]
    heads = al_ref.shape[1] // 2
    for c in range(nch):
        h = c * heads // nch
        scale = jnp.exp(al_ref[:, h:h + 1] - m_ref[0:1, h:h + 1])
        y_ref[c, :, :] = xs_ref[:, c * CHUNK:(c + 1) * CHUNK] * scale
    a4 = jnp.exp(al_ref[:, 0:heads] - m_ref[0:1, 0:heads])
    ap_ref[...] = jnp.concatenate(
        [a4, jnp.zeros((b, CHUNK - heads), F32)], axis=1)


def _z_kernel(agga_ref, al_ref, m_ref, z_ref):
    heads = al_ref.shape[1] // 2
    b4 = jnp.exp(al_ref[:, heads:2 * heads] - m_ref[0:1, heads:2 * heads])
    z_ref[...] = jnp.sum(b4 * agga_ref[:, 0:heads], axis=0, keepdims=True)


def _mix_kernel(agg_ref, al_ref, m_ref, z_ref, out_ref):
    heads = al_ref.shape[1] // 2
    halves = agg_ref.shape[0] // heads
    b = al_ref.shape[0]
    for half in range(halves):
        acc = jnp.zeros((b, CHUNK), F32)
        for h in range(heads):
            wgt = jnp.exp(al_ref[:, heads + h:heads + h + 1]
                          - m_ref[0:1, heads + h:heads + h + 1])
            wgt = wgt / (heads * z_ref[0:1, h:h + 1])
            acc = acc + wgt * agg_ref[halves * h + half, :, :]
        out_ref[:, half * CHUNK:(half + 1) * CHUNK] = acc


def _make_sc_kernel(n, e, nch):
    e_t = e // NS
    nw = e_t // W
    rpt = 8 * ((n // NS) // 8 + 1)
    tail_base = (NS - 1) * rpt
    tail_rows = n - tail_base
    mesh = plsc.VectorSubcoreMesh(core_axis_name="c", subcore_axis_name="s",
                                  num_cores=NC, num_subcores=NS)

    @functools.partial(
        pl.kernel,
        out_type=(jax.ShapeDtypeStruct((nch, n, CHUNK), F32),
                  jax.ShapeDtypeStruct((n, CHUNK), F32)),
        mesh=mesh,
        scratch_types=[
            pltpu.VMEM((1, W), jnp.int32),
            pltpu.VMEM((1, W), jnp.int32),
            pltpu.VMEM((W, CHUNK), F32),
            pltpu.VMEM_SHARED((n, CHUNK), F32),
        ],
    )
    def sc_kernel(y_hbm, ap_hbm, row_hbm, col_hbm, z128_hbm,
                  agg_hbm, agga_hbm, rbuf, cbuf, gbuf, acc):
        c_id = lax.axis_index("c")
        s_id = lax.axis_index("s")
        ebase = s_id * e_t

        def rows_copy(src, dst):
            start = pl.multiple_of(s_id * rpt, 8)

            @pl.when(s_id < NS - 1)
            def _():
                pltpu.sync_copy(src.at[pl.ds(start, rpt)],
                                dst.at[pl.ds(start, rpt)])

            @pl.when(s_id == NS - 1)
            def _():
                pltpu.sync_copy(src.at[pl.ds(tail_base, tail_rows)],
                                dst.at[pl.ds(tail_base, tail_rows)])

        def edge_pass(table, buf, accum):
            @pl.loop(0, nw)
            def _(w):
                base = pl.multiple_of(ebase + w * W, 8)
                pltpu.sync_copy(row_hbm.at[pl.ds(base, W)], rbuf.at[0])
                pltpu.sync_copy(col_hbm.at[pl.ds(base, W)], cbuf.at[0])
                pltpu.sync_copy(table.at[rbuf.at[0]], buf)
                pltpu.sync_copy(buf, accum.at[cbuf.at[0]], add=True)

        def do_chunk(ch):
            rows_copy(z128_hbm, acc)
            plsc.subcore_barrier()
            edge_pass(y_hbm.at[ch], gbuf, acc)
            plsc.subcore_barrier()
            rows_copy(acc, agg_hbm.at[ch])
            plsc.subcore_barrier()

        per_core = nch // NC
        for core in range(NC):
            @pl.when(c_id == core)
            def _():
                for j in range(per_core):
                    do_chunk(core * per_core + j)

        @pl.when(c_id == NC - 1)
        def _():
            rows_copy(z128_hbm, acc)
            plsc.subcore_barrier()
            edge_pass(ap_hbm, gbuf, acc)
            plsc.subcore_barrier()
            rows_copy(acc, agga_hbm)

    return sc_kernel


def kernel(x, edge_index, W_src, W_dst, att_src, att_dst):
    n, d = x.shape
    e = edge_index.shape[1]
    heads, c_out = att_src.shape[1], att_src.shape[2]
    hc = heads * c_out
    nch = hc // CHUNK
    blk = 1000
    nblk = n // blk

    ws_t = W_src.T
    wd_t = W_dst.T
    eye = jnp.repeat(jnp.eye(heads, dtype=F32), c_out, axis=0)
    as_exp = eye * att_src.reshape(hc, 1)
    ad_exp = eye * att_dst.reshape(hc, 1)
    row = edge_index[0]
    col = edge_index[1]
    z128 = jnp.zeros((n, CHUNK), F32)

    grid_a = (nblk,)
    xs, al, m = pl.pallas_call(
        _proj_kernel,
        grid=grid_a,
        in_specs=[
            pl.BlockSpec((blk, d), lambda i: (i, 0)),
            pl.BlockSpec((d, hc), lambda i: (0, 0)),
            pl.BlockSpec((d, hc), lambda i: (0, 0)),
            pl.BlockSpec((hc, heads), lambda i: (0, 0)),
            pl.BlockSpec((hc, heads), lambda i: (0, 0)),
        ],
        out_specs=[
            pl.BlockSpec((blk, hc), lambda i: (i, 0)),
            pl.BlockSpec((blk, 2 * heads), lambda i: (i, 0)),
            pl.BlockSpec((1, 2 * heads), lambda i: (0, 0)),
        ],
        out_shape=[
            jax.ShapeDtypeStruct((n, hc), F32),
            jax.ShapeDtypeStruct((n, 2 * heads), F32),
            jax.ShapeDtypeStruct((1, 2 * heads), F32),
        ],
        scratch_shapes=[
            pltpu.VMEM((d, 2 * heads), F32),
            pltpu.VMEM((1, 2 * heads), F32),
        ],
    )(x, ws_t, wd_t, as_exp, ad_exp)

    y, ap = pl.pallas_call(
        _scale_kernel,
        grid=grid_a,
        in_specs=[
            pl.BlockSpec((blk, hc), lambda i: (i, 0)),
            pl.BlockSpec((blk, 2 * heads), lambda i: (i, 0)),
            pl.BlockSpec((1, 2 * heads), lambda i: (0, 0)),
        ],
        out_specs=[
            pl.BlockSpec((nch, blk, CHUNK), lambda i: (0, i, 0)),
            pl.BlockSpec((blk, CHUNK), lambda i: (i, 0)),
        ],
        out_shape=[
            jax.ShapeDtypeStruct((nch, n, CHUNK), F32),
            jax.ShapeDtypeStruct((n, CHUNK), F32),
        ],
    )(xs, al, m)

    agg, agga = _make_sc_kernel(n, e, nch)(y, ap, row, col, z128)

    z = pl.pallas_call(
        _z_kernel,
        grid=(1,),
        in_specs=[
            pl.BlockSpec((n, CHUNK), lambda i: (0, 0)),
            pl.BlockSpec((n, 2 * heads), lambda i: (0, 0)),
            pl.BlockSpec((1, 2 * heads), lambda i: (0, 0)),
        ],
        out_specs=[pl.BlockSpec((1, heads), lambda i: (0, 0))],
        out_shape=[jax.ShapeDtypeStruct((1, heads), F32)],
    )(agga, al, m)[0]

    out = pl.pallas_call(
        _mix_kernel,
        grid=grid_a,
        in_specs=[
            pl.BlockSpec((nch, blk, CHUNK), lambda i: (0, i, 0)),
            pl.BlockSpec((blk, 2 * heads), lambda i: (i, 0)),
            pl.BlockSpec((1, 2 * heads), lambda i: (0, 0)),
            pl.BlockSpec((1, heads), lambda i: (0, 0)),
        ],
        out_specs=pl.BlockSpec((blk, c_out), lambda i: (i, 0)),
        out_shape=jax.ShapeDtypeStruct((n, c_out), F32),
    )(agg, al, m, z)
    return out


# speedup vs baseline: 31.8218x; 2.5755x over previous
import functools

import jax
import jax.numpy as jnp
from jax import lax
from jax.experimental import pallas as pl
from jax.experimental.pallas import tpu as pltpu
from jax.experimental.pallas import tpu_sc as plsc

F32 = jnp.float32
NC = 2
NS = 16
CHUNK = 128
W = 125


def _proj_kernel(x_ref, ws_ref, wd_ref, as_exp_ref, ad_exp_ref, xs_ref,
                 al_ref, m_ref, v_scr, m_scr):
    i = pl.program_id(0)

    @pl.when(i == 0)
    def _():
        v_scr[...] = jnp.concatenate(
            [jnp.dot(ws_ref[...], as_exp_ref[...],
                     preferred_element_type=F32),
             jnp.dot(wd_ref[...], ad_exp_ref[...],
                     preferred_element_type=F32)], axis=1)
        m_scr[...] = jnp.full(m_scr.shape, -jnp.inf, F32)

    xb = x_ref[...]
    xs = jnp.dot(xb, ws_ref[...], preferred_element_type=F32)
    xs_ref[...] = xs
    al = jnp.dot(xb, v_scr[...], preferred_element_type=F32)
    al_ref[...] = al
    m_scr[...] = jnp.maximum(m_scr[...], jnp.max(al, axis=0, keepdims=True))
    m_ref[...] = m_scr[...]


def _scale_kernel(xs_ref, al_ref, m_ref, y_ref, ap_ref):
    nch = y_ref.shape[0]
    b = xs_ref.shape[0]
    heads = al_ref.shape[1] // 2
    for c in range(nch):
        h = c * heads // nch
        scale = jnp.exp(al_ref[:, h:h + 1] - m_ref[0:1, h:h + 1])
        y_ref[c, :, :] = xs_ref[:, c * CHUNK:(c + 1) * CHUNK] * scale
    a4 = jnp.exp(al_ref[:, 0:heads] - m_ref[0:1, 0:heads])
    ap_ref[...] = jnp.concatenate(
        [a4, jnp.zeros((b, CHUNK - heads), F32)], axis=1)


def _z_kernel(agga_ref, al_ref, m_ref, z_ref):
    heads = al_ref.shape[1] // 2
    b4 = jnp.exp(al_ref[:, heads:2 * heads] - m_ref[0:1, heads:2 * heads])
    z_ref[...] = jnp.sum(b4 * agga_ref[:, 0:heads], axis=0, keepdims=True)


def _mix_kernel(agg_ref, al_ref, m_ref, z_ref, out_ref):
    heads = al_ref.shape[1] // 2
    halves = agg_ref.shape[0] // heads
    b = al_ref.shape[0]
    for half in range(halves):
        acc = jnp.zeros((b, CHUNK), F32)
        for h in range(heads):
            wgt = jnp.exp(al_ref[:, heads + h:heads + h + 1]
                          - m_ref[0:1, heads + h:heads + h + 1])
            wgt = wgt / (heads * z_ref[0:1, h:h + 1])
            acc = acc + wgt * agg_ref[halves * h + half, :, :]
        out_ref[:, half * CHUNK:(half + 1) * CHUNK] = acc


def _make_sc_kernel(n, e, nch):
    e_t = e // NS
    nw = e_t // W
    rpt = 8 * ((n // NS) // 8 + 1)
    tail_base = (NS - 1) * rpt
    tail_rows = n - tail_base
    mesh = plsc.VectorSubcoreMesh(core_axis_name="c", subcore_axis_name="s",
                                  num_cores=NC, num_subcores=NS)

    @functools.partial(
        pl.kernel,
        out_type=(jax.ShapeDtypeStruct((nch, n, CHUNK), F32),
                  jax.ShapeDtypeStruct((n, CHUNK), F32)),
        mesh=mesh,
        scratch_types=[
            pltpu.VMEM((nw, W), jnp.int32),
            pltpu.VMEM((2, W), jnp.int32),
            pltpu.VMEM((W, CHUNK), F32),
            pltpu.VMEM((W, CHUNK), F32),
            pltpu.VMEM_SHARED((n, CHUNK), F32),
            pltpu.SemaphoreType.DMA,
            pltpu.SemaphoreType.DMA,
            pltpu.SemaphoreType.DMA,
            pltpu.SemaphoreType.DMA,
        ],
    )
    def sc_kernel(y_hbm, ap_hbm, row_hbm, col_hbm, z128_hbm,
                  agg_hbm, agga_hbm, rbuf, cbuf, g0, g1, acc,
                  sem0, sem1, semc0, semc1):
        c_id = lax.axis_index("c")
        s_id = lax.axis_index("s")

        pltpu.sync_copy(row_hbm.at[s_id], rbuf)

        def rows_copy(src, dst):
            start = pl.multiple_of(s_id * rpt, 8)

            @pl.when(s_id < NS - 1)
            def _():
                pltpu.sync_copy(src.at[pl.ds(start, rpt)],
                                dst.at[pl.ds(start, rpt)])

            @pl.when(s_id == NS - 1)
            def _():
                pltpu.sync_copy(src.at[pl.ds(tail_base, tail_rows)],
                                dst.at[pl.ds(tail_base, tail_rows)])

        def edge_pass(table, accum):
            cb = s_id * nw
            pltpu.async_copy(col_hbm.at[cb], cbuf.at[pl.ds(0, 1)], semc0)
            pltpu.async_copy(table.at[rbuf.at[0]], g0, sem0)

            @pl.loop(0, nw - 1, step=2)
            def _(w):
                pltpu.async_copy(table.at[rbuf.at[w + 1]], g1, sem1)
                pltpu.async_copy(col_hbm.at[cb + w + 1],
                                 cbuf.at[pl.ds(1, 1)], semc1)
                pltpu.make_async_copy(col_hbm.at[cb + w],
                                      cbuf.at[pl.ds(0, 1)], semc0).wait()
                pltpu.make_async_copy(table.at[rbuf.at[w]], g0, sem0).wait()
                pltpu.sync_copy(g0, accum.at[cbuf.at[0]], add=True)

                @pl.when(w + 2 < nw)
                def _():
                    pltpu.async_copy(table.at[rbuf.at[w + 2]], g0, sem0)
                    pltpu.async_copy(col_hbm.at[cb + w + 2],
                                     cbuf.at[pl.ds(0, 1)], semc0)

                pltpu.make_async_copy(col_hbm.at[cb + w + 1],
                                      cbuf.at[pl.ds(1, 1)], semc1).wait()
                pltpu.make_async_copy(table.at[rbuf.at[w + 1]], g1,
                                      sem1).wait()
                pltpu.sync_copy(g1, accum.at[cbuf.at[1]], add=True)

            if nw % 2:
                pltpu.make_async_copy(col_hbm.at[cb + nw - 1],
                                      cbuf.at[pl.ds(0, 1)], semc0).wait()
                pltpu.make_async_copy(table.at[rbuf.at[nw - 1]], g0,
                                      sem0).wait()
                pltpu.sync_copy(g0, accum.at[cbuf.at[0]], add=True)

        def do_chunk(ch):
            rows_copy(z128_hbm, acc)
            plsc.subcore_barrier()
            edge_pass(y_hbm.at[ch], acc)
            plsc.subcore_barrier()
            rows_copy(acc, agg_hbm.at[ch])
            plsc.subcore_barrier()

        per_core = nch // NC
        for core in range(NC):
            @pl.when(c_id == core)
            def _():
                for j in range(per_core):
                    do_chunk(core * per_core + j)

        @pl.when(c_id == NC - 1)
        def _():
            rows_copy(z128_hbm, acc)
            plsc.subcore_barrier()
            edge_pass(ap_hbm, acc)
            plsc.subcore_barrier()
            rows_copy(acc, agga_hbm)

    return sc_kernel


def kernel(x, edge_index, W_src, W_dst, att_src, att_dst):
    n, d = x.shape
    e = edge_index.shape[1]
    heads, c_out = att_src.shape[1], att_src.shape[2]
    hc = heads * c_out
    nch = hc // CHUNK
    blk = 1000
    nblk = n // blk

    ws_t = W_src.T
    wd_t = W_dst.T
    eye = jnp.repeat(jnp.eye(heads, dtype=F32), c_out, axis=0)
    as_exp = eye * att_src.reshape(hc, 1)
    ad_exp = eye * att_dst.reshape(hc, 1)
    e_t = e // NS
    row = edge_index[0].reshape(NS, e_t // W, W)
    col = edge_index[1].reshape(NS * (e_t // W), 1, W)
    z128 = jnp.zeros((n, CHUNK), F32)

    grid_a = (nblk,)
    xs, al, m = pl.pallas_call(
        _proj_kernel,
        grid=grid_a,
        in_specs=[
            pl.BlockSpec((blk, d), lambda i: (i, 0)),
            pl.BlockSpec((d, hc), lambda i: (0, 0)),
            pl.BlockSpec((d, hc), lambda i: (0, 0)),
            pl.BlockSpec((hc, heads), lambda i: (0, 0)),
            pl.BlockSpec((hc, heads), lambda i: (0, 0)),
        ],
        out_specs=[
            pl.BlockSpec((blk, hc), lambda i: (i, 0)),
            pl.BlockSpec((blk, 2 * heads), lambda i: (i, 0)),
            pl.BlockSpec((1, 2 * heads), lambda i: (0, 0)),
        ],
        out_shape=[
            jax.ShapeDtypeStruct((n, hc), F32),
            jax.ShapeDtypeStruct((n, 2 * heads), F32),
            jax.ShapeDtypeStruct((1, 2 * heads), F32),
        ],
        scratch_shapes=[
            pltpu.VMEM((d, 2 * heads), F32),
            pltpu.VMEM((1, 2 * heads), F32),
        ],
    )(x, ws_t, wd_t, as_exp, ad_exp)

    y, ap = pl.pallas_call(
        _scale_kernel,
        grid=grid_a,
        in_specs=[
            pl.BlockSpec((blk, hc), lambda i: (i, 0)),
            pl.BlockSpec((blk, 2 * heads), lambda i: (i, 0)),
            pl.BlockSpec((1, 2 * heads), lambda i: (0, 0)),
        ],
        out_specs=[
            pl.BlockSpec((nch, blk, CHUNK), lambda i: (0, i, 0)),
            pl.BlockSpec((blk, CHUNK), lambda i: (i, 0)),
        ],
        out_shape=[
            jax.ShapeDtypeStruct((nch, n, CHUNK), F32),
            jax.ShapeDtypeStruct((n, CHUNK), F32),
        ],
    )(xs, al, m)

    agg, agga = _make_sc_kernel(n, e, nch)(y, ap, row, col, z128)

    z = pl.pallas_call(
        _z_kernel,
        grid=(1,),
        in_specs=[
            pl.BlockSpec((n, CHUNK), lambda i: (0, 0)),
            pl.BlockSpec((n, 2 * heads), lambda i: (0, 0)),
            pl.BlockSpec((1, 2 * heads), lambda i: (0, 0)),
        ],
        out_specs=[pl.BlockSpec((1, heads), lambda i: (0, 0))],
        out_shape=[jax.ShapeDtypeStruct((1, heads), F32)],
    )(agga, al, m)[0]

    out = pl.pallas_call(
        _mix_kernel,
        grid=grid_a,
        in_specs=[
            pl.BlockSpec((nch, blk, CHUNK), lambda i: (0, i, 0)),
            pl.BlockSpec((blk, 2 * heads), lambda i: (i, 0)),
            pl.BlockSpec((1, 2 * heads), lambda i: (0, 0)),
            pl.BlockSpec((1, heads), lambda i: (0, 0)),
        ],
        out_specs=pl.BlockSpec((blk, c_out), lambda i: (i, 0)),
        out_shape=jax.ShapeDtypeStruct((n, c_out), F32),
    )(agg, al, m, z)
    return out


# speedup vs baseline: 33.8840x; 1.0648x over previous
import functools

import jax
import jax.numpy as jnp
from jax import lax
from jax.experimental import pallas as pl
from jax.experimental.pallas import tpu as pltpu
from jax.experimental.pallas import tpu_sc as plsc

F32 = jnp.float32
NC = 2
NS = 16
CHUNK = 128
W = 125


def _proj_kernel(x_ref, ws_ref, wd_ref, as_exp_ref, ad_exp_ref, xs_ref,
                 al_ref, m_ref, v_scr, m_scr):
    i = pl.program_id(0)

    @pl.when(i == 0)
    def _():
        v_scr[...] = jnp.concatenate(
            [jnp.dot(ws_ref[...], as_exp_ref[...],
                     preferred_element_type=F32),
             jnp.dot(wd_ref[...], ad_exp_ref[...],
                     preferred_element_type=F32)], axis=1)
        m_scr[...] = jnp.full(m_scr.shape, -jnp.inf, F32)

    xb = x_ref[...]
    xs = jnp.dot(xb, ws_ref[...], preferred_element_type=F32)
    xs_ref[...] = xs
    al = jnp.dot(xb, v_scr[...], preferred_element_type=F32)
    al_ref[...] = al
    m_scr[...] = jnp.maximum(m_scr[...], jnp.max(al, axis=0, keepdims=True))
    m_ref[...] = m_scr[...]


def _scale_kernel(xs_ref, al_ref, m_ref, y_ref, ap_ref):
    nch = y_ref.shape[0]
    b = xs_ref.shape[0]
    heads = al_ref.shape[1] // 2
    for c in range(nch):
        h = c * heads // nch
        scale = jnp.exp(al_ref[:, h:h + 1] - m_ref[0:1, h:h + 1])
        y_ref[c, :, :] = xs_ref[:, c * CHUNK:(c + 1) * CHUNK] * scale
    a4 = jnp.exp(al_ref[:, 0:heads] - m_ref[0:1, 0:heads])
    ap_ref[...] = jnp.concatenate(
        [a4, jnp.zeros((b, CHUNK - heads), F32)], axis=1)


def _z_kernel(agga_ref, al_ref, m_ref, z_ref):
    heads = al_ref.shape[1] // 2
    b4 = jnp.exp(al_ref[:, heads:2 * heads] - m_ref[0:1, heads:2 * heads])
    agga = agga_ref[0, :, 0:heads]
    for k in range(1, agga_ref.shape[0]):
        agga = agga + agga_ref[k, :, 0:heads]
    z_ref[...] = jnp.sum(b4 * agga, axis=0, keepdims=True)


def _mix_kernel(agg_ref, al_ref, m_ref, z_ref, out_ref):
    heads = al_ref.shape[1] // 2
    halves = agg_ref.shape[0] // heads
    b = al_ref.shape[0]
    for half in range(halves):
        acc = jnp.zeros((b, CHUNK), F32)
        for h in range(heads):
            wgt = jnp.exp(al_ref[:, heads + h:heads + h + 1]
                          - m_ref[0:1, heads + h:heads + h + 1])
            wgt = wgt / (heads * z_ref[0:1, h:h + 1])
            acc = acc + wgt * agg_ref[halves * h + half, :, :]
        out_ref[:, half * CHUNK:(half + 1) * CHUNK] = acc


def _make_sc_kernel(n, e, nch):
    e_t = e // NS
    nw = e_t // W
    rpt = 8 * ((n // NS) // 8 + 1)
    tail_base = (NS - 1) * rpt
    tail_rows = n - tail_base
    mesh = plsc.VectorSubcoreMesh(core_axis_name="c", subcore_axis_name="s",
                                  num_cores=NC, num_subcores=NS)

    @functools.partial(
        pl.kernel,
        out_type=(jax.ShapeDtypeStruct((nch, n, CHUNK), F32),
                  jax.ShapeDtypeStruct((NC, n, CHUNK), F32)),
        mesh=mesh,
        scratch_types=[
            pltpu.VMEM((nw, W), jnp.int32),
            pltpu.VMEM((2, W), jnp.int32),
            pltpu.VMEM((W, CHUNK), F32),
            pltpu.VMEM((W, CHUNK), F32),
            pltpu.VMEM_SHARED((n, CHUNK), F32),
            pltpu.SemaphoreType.DMA,
            pltpu.SemaphoreType.DMA,
            pltpu.SemaphoreType.DMA,
            pltpu.SemaphoreType.DMA,
        ],
    )
    def sc_kernel(y_hbm, ap_hbm, row_hbm, col_hbm, z128_hbm,
                  agg_hbm, agga_hbm, rbuf, cbuf, g0, g1, acc,
                  sem0, sem1, semc0, semc1):
        c_id = lax.axis_index("c")
        s_id = lax.axis_index("s")

        pltpu.sync_copy(row_hbm.at[s_id], rbuf)

        def rows_copy(src, dst):
            start = pl.multiple_of(s_id * rpt, 8)

            @pl.when(s_id < NS - 1)
            def _():
                pltpu.sync_copy(src.at[pl.ds(start, rpt)],
                                dst.at[pl.ds(start, rpt)])

            @pl.when(s_id == NS - 1)
            def _():
                pltpu.sync_copy(src.at[pl.ds(tail_base, tail_rows)],
                                dst.at[pl.ds(tail_base, tail_rows)])

        def edge_pass(table, accum, w_lo, w_hi):
            cb = s_id * nw
            pltpu.async_copy(col_hbm.at[cb + w_lo], cbuf.at[pl.ds(0, 1)],
                             semc0)
            pltpu.async_copy(table.at[rbuf.at[w_lo]], g0, sem0)

            @pl.loop(w_lo, w_hi - 1, step=2)
            def _(w):
                pltpu.async_copy(table.at[rbuf.at[w + 1]], g1, sem1)
                pltpu.async_copy(col_hbm.at[cb + w + 1],
                                 cbuf.at[pl.ds(1, 1)], semc1)
                pltpu.make_async_copy(col_hbm.at[cb + w],
                                      cbuf.at[pl.ds(0, 1)], semc0).wait()
                pltpu.make_async_copy(table.at[rbuf.at[w]], g0, sem0).wait()
                pltpu.sync_copy(g0, accum.at[cbuf.at[0]], add=True)

                @pl.when(w + 2 < w_hi)
                def _():
                    pltpu.async_copy(table.at[rbuf.at[w + 2]], g0, sem0)
                    pltpu.async_copy(col_hbm.at[cb + w + 2],
                                     cbuf.at[pl.ds(0, 1)], semc0)

                pltpu.make_async_copy(col_hbm.at[cb + w + 1],
                                      cbuf.at[pl.ds(1, 1)], semc1).wait()
                pltpu.make_async_copy(table.at[rbuf.at[w + 1]], g1,
                                      sem1).wait()
                pltpu.sync_copy(g1, accum.at[cbuf.at[1]], add=True)

            if (w_hi - w_lo) % 2:
                pltpu.make_async_copy(col_hbm.at[cb + w_hi - 1],
                                      cbuf.at[pl.ds(0, 1)], semc0).wait()
                pltpu.make_async_copy(table.at[rbuf.at[w_hi - 1]], g0,
                                      sem0).wait()
                pltpu.sync_copy(g0, accum.at[cbuf.at[0]], add=True)

        def do_chunk(ch):
            rows_copy(z128_hbm, acc)
            plsc.subcore_barrier()
            edge_pass(y_hbm.at[ch], acc, 0, nw)
            plsc.subcore_barrier()
            rows_copy(acc, agg_hbm.at[ch])
            plsc.subcore_barrier()

        per_core = nch // NC
        for core in range(NC):
            @pl.when(c_id == core)
            def _():
                for j in range(per_core):
                    do_chunk(core * per_core + j)
                rows_copy(z128_hbm, acc)
                plsc.subcore_barrier()
                edge_pass(ap_hbm, acc, core * nw // NC,
                          (core + 1) * nw // NC)
                plsc.subcore_barrier()
                rows_copy(acc, agga_hbm.at[core])

    return sc_kernel


def kernel(x, edge_index, W_src, W_dst, att_src, att_dst):
    n, d = x.shape
    e = edge_index.shape[1]
    heads, c_out = att_src.shape[1], att_src.shape[2]
    hc = heads * c_out
    nch = hc // CHUNK
    blk = 1000
    nblk = n // blk

    ws_t = W_src.T
    wd_t = W_dst.T
    eye = jnp.repeat(jnp.eye(heads, dtype=F32), c_out, axis=0)
    as_exp = eye * att_src.reshape(hc, 1)
    ad_exp = eye * att_dst.reshape(hc, 1)
    e_t = e // NS
    row = edge_index[0].reshape(NS, e_t // W, W)
    col = edge_index[1].reshape(NS * (e_t // W), 1, W)
    z128 = jnp.zeros((n, CHUNK), F32)

    grid_a = (nblk,)
    xs, al, m = pl.pallas_call(
        _proj_kernel,
        grid=grid_a,
        in_specs=[
            pl.BlockSpec((blk, d), lambda i: (i, 0)),
            pl.BlockSpec((d, hc), lambda i: (0, 0)),
            pl.BlockSpec((d, hc), lambda i: (0, 0)),
            pl.BlockSpec((hc, heads), lambda i: (0, 0)),
            pl.BlockSpec((hc, heads), lambda i: (0, 0)),
        ],
        out_specs=[
            pl.BlockSpec((blk, hc), lambda i: (i, 0)),
            pl.BlockSpec((blk, 2 * heads), lambda i: (i, 0)),
            pl.BlockSpec((1, 2 * heads), lambda i: (0, 0)),
        ],
        out_shape=[
            jax.ShapeDtypeStruct((n, hc), F32),
            jax.ShapeDtypeStruct((n, 2 * heads), F32),
            jax.ShapeDtypeStruct((1, 2 * heads), F32),
        ],
        scratch_shapes=[
            pltpu.VMEM((d, 2 * heads), F32),
            pltpu.VMEM((1, 2 * heads), F32),
        ],
    )(x, ws_t, wd_t, as_exp, ad_exp)

    y, ap = pl.pallas_call(
        _scale_kernel,
        grid=grid_a,
        in_specs=[
            pl.BlockSpec((blk, hc), lambda i: (i, 0)),
            pl.BlockSpec((blk, 2 * heads), lambda i: (i, 0)),
            pl.BlockSpec((1, 2 * heads), lambda i: (0, 0)),
        ],
        out_specs=[
            pl.BlockSpec((nch, blk, CHUNK), lambda i: (0, i, 0)),
            pl.BlockSpec((blk, CHUNK), lambda i: (i, 0)),
        ],
        out_shape=[
            jax.ShapeDtypeStruct((nch, n, CHUNK), F32),
            jax.ShapeDtypeStruct((n, CHUNK), F32),
        ],
    )(xs, al, m)

    agg, agga = _make_sc_kernel(n, e, nch)(y, ap, row, col, z128)

    z = pl.pallas_call(
        _z_kernel,
        grid=(1,),
        in_specs=[
            pl.BlockSpec((NC, n, CHUNK), lambda i: (0, 0, 0)),
            pl.BlockSpec((n, 2 * heads), lambda i: (0, 0)),
            pl.BlockSpec((1, 2 * heads), lambda i: (0, 0)),
        ],
        out_specs=[pl.BlockSpec((1, heads), lambda i: (0, 0))],
        out_shape=[jax.ShapeDtypeStruct((1, heads), F32)],
    )(agga, al, m)[0]

    out = pl.pallas_call(
        _mix_kernel,
        grid=grid_a,
        in_specs=[
            pl.BlockSpec((nch, blk, CHUNK), lambda i: (0, i, 0)),
            pl.BlockSpec((blk, 2 * heads), lambda i: (i, 0)),
            pl.BlockSpec((1, 2 * heads), lambda i: (0, 0)),
            pl.BlockSpec((1, heads), lambda i: (0, 0)),
        ],
        out_specs=pl.BlockSpec((blk, c_out), lambda i: (i, 0)),
        out_shape=jax.ShapeDtypeStruct((n, c_out), F32),
    )(agg, al, m, z)
    return out
